# baseline (speedup 1.0000x reference)
import functools

import jax
import jax.numpy as jnp
from jax import lax
from jax.experimental import pallas as pl
from jax.experimental.pallas import tpu as pltpu
from jax.experimental.pallas import tpu_sc as plsc

NC = 2
NS = 16
NW = NC * NS
L = 16

B = 131072
D = 64
N = 1000000

B_PER_W = B // NW
R = N // NW
SPAD = 31264
ICH = 2048
N_ICH = B // ICH
CAP = 16384
BLK = 256
GCH = 256

_mesh = plsc.VectorSubcoreMesh(
    core_axis_name="c", subcore_axis_name="s", num_cores=NC, num_subcores=NS
)

_i32 = jnp.int32


@functools.partial(
    pl.kernel,
    out_type=jax.ShapeDtypeStruct((B, D), jnp.float32),
    mesh=_mesh,
    compiler_params=pltpu.CompilerParams(
        use_tc_tiling_on_sc=False, needs_layout_passes=False
    ),
    scratch_types=[
        pltpu.VMEM((SPAD,), _i32),
        pltpu.VMEM((ICH,), _i32),
        pltpu.VMEM((CAP,), _i32),
        pltpu.VMEM((CAP,), _i32),
        pltpu.VMEM((BLK, D), jnp.float32),
        pltpu.VMEM((GCH, D), jnp.float32),
        pltpu.VMEM((GCH,), _i32),
        pltpu.SemaphoreType.DMA,
        pltpu.SemaphoreType.DMA,
    ],
)
def _sc_pool(mem, idx, val, new_mem, pulled,
             stamp, idx_v, pos_list, row_list, sbuf, gbuf, gidx,
             sem_a, sem_b):
    wid = lax.axis_index("s") * NC + lax.axis_index("c")
    lo = wid * R
    lanes = lax.iota(_i32, L)
    neg1 = jnp.full((L,), -1, _i32)

    def init_body(j, _):
        stamp[pl.ds(j * L, L)] = neg1
        return 0
    lax.fori_loop(0, SPAD // L, init_body, 0)

    def scan_chunk(c, _):
        pltpu.sync_copy(idx.at[pl.ds(c * ICH, ICH)], idx_v)

        def scan_vreg(j, _):
            v = idx_v[pl.ds(j * L, L)]
            local = v - lo
            pos = (c * ICH + j * L) + lanes
            own = (local >= 0) & (local < R)
            _, lastocc = plsc.scan_count(v, mask=own)
            m = own & lastocc
            plsc.store_scatter(stamp, [local], pos, mask=m)
            return 0

        lax.fori_loop(0, ICH // L, scan_vreg, 0)
        return 0

    lax.fori_loop(0, N_ICH, scan_chunk, 0)

    def extract(j, cursor):
        s = stamp[pl.ds(j * L, L)]
        m = s >= 0
        cnt = jnp.sum(m.astype(_i32))
        cur = jnp.minimum(cursor, CAP - L)
        rows = (lo + j * L) + lanes
        plsc.store_compressed(pos_list.at[pl.ds(cur, L)], s, mask=m)
        plsc.store_compressed(row_list.at[pl.ds(cur, L)], rows, mask=m)
        return cursor + cnt

    count = lax.fori_loop(0, SPAD // L, extract, jnp.asarray(0, _i32))
    count = jnp.minimum(count, CAP)

    w0 = pos_list[pl.ds(0, L)]
    r0 = row_list[pl.ds(0, L)]
    lane0 = (lanes == 0).astype(_i32)
    p0 = jnp.sum(jnp.where(lanes == 0, w0, 0))
    q0 = jnp.sum(jnp.where(lanes == 0, r0, 0))
    padp = jnp.full((L,), 1, _i32) * p0
    padq = jnp.full((L,), 1, _i32) * q0
    del lane0

    nblocks = (count + BLK - 1) // BLK
    cend = nblocks * BLK
    j0 = (count // L) * L

    def pad_body(t, _):
        base = t * L
        keep = (base + lanes) < count
        wp = jnp.where(keep, pos_list[pl.ds(base, L)], padp)
        rp = jnp.where(keep, row_list[pl.ds(base, L)], padq)
        pos_list[pl.ds(base, L)] = wp
        row_list[pl.ds(base, L)] = rp
        return 0

    lax.fori_loop(j0 // L, cend // L, pad_body, 0)

    def move_block(b, _):
        pltpu.async_copy(
            val.at[pos_list.at[pl.ds(b * BLK, BLK)]], sbuf, sem_a
        ).wait()
        pltpu.async_copy(
            sbuf, new_mem.at[row_list.at[pl.ds(b * BLK, BLK)]], sem_b
        ).wait()
        return 0

    lax.fori_loop(0, nblocks, move_block, 0)

    base = wid * B_PER_W
    for g in range(B_PER_W // GCH):
        off = base + g * GCH
        pltpu.sync_copy(idx.at[pl.ds(off, GCH)], gidx)
        pltpu.async_copy(mem.at[gidx], gbuf, sem_a).wait()
        pltpu.sync_copy(gbuf, pulled.at[pl.ds(off, GCH)])


def kernel(mem, idx, val):
    new_mem_ref = jax.new_ref(mem)
    pulled = _sc_pool(mem, idx, val, new_mem_ref)
    return pulled, new_mem_ref[...]


# speedup vs baseline: 2.0179x; 2.0179x over previous
import functools

import jax
import jax.numpy as jnp
from jax import lax
from jax.experimental import pallas as pl
from jax.experimental.pallas import tpu as pltpu
from jax.experimental.pallas import tpu_sc as plsc

NC = 2
NS = 16
NW = NC * NS
L = 16

B = 131072
D = 64
N = 1000000

B_PER_W = B // NW
R = N // NW
SPAD = 31264
ICH = 2048
N_ICH = B // ICH
CAP = 8192
CB = 128
NB_FULL = R // CB
TAIL = R - NB_FULL * CB

_mesh = plsc.VectorSubcoreMesh(
    core_axis_name="c", subcore_axis_name="s", num_cores=NC, num_subcores=NS
)

_i32 = jnp.int32
_f32 = jnp.float32


@functools.partial(
    pl.kernel,
    out_type=(
        jax.ShapeDtypeStruct((B, D), _f32),
        jax.ShapeDtypeStruct((N, D), _f32),
    ),
    mesh=_mesh,
    compiler_params=pltpu.CompilerParams(
        use_tc_tiling_on_sc=False, needs_layout_passes=False
    ),
    scratch_types=[
        pltpu.VMEM((SPAD,), _i32),
        pltpu.VMEM((ICH,), _i32),
        pltpu.VMEM((CAP,), _i32),
        pltpu.VMEM((CAP,), _i32),
        pltpu.VMEM((CB, D), _f32),
        pltpu.VMEM((CB, D), _f32),
        pltpu.VMEM((CB, D), _f32),
        pltpu.VMEM((CB, D), _f32),
        pltpu.VMEM((4 * CB,), _i32),
        pltpu.SemaphoreType.DMA,
        pltpu.SemaphoreType.DMA,
    ],
)
def _sc_pool(mem, idx, val, pulled, new_mem,
             stamp, idx_v, pos_list, row_list, cb0, cb1, cb2, cb3, gidx,
             isem, osem):
    cbs = (cb0, cb1, cb2, cb3)
    wid = lax.axis_index("s") * NC + lax.axis_index("c")
    lo = wid * R
    lanes = lax.iota(_i32, L)
    neg1 = jnp.full((L,), -1, _i32)

    def init_body(j, _):
        stamp[pl.ds(j * L, L)] = neg1
        return 0
    lax.fori_loop(0, SPAD // L, init_body, 0)

    def scan_chunk(c, _):
        pltpu.sync_copy(idx.at[pl.ds(c * ICH, ICH)], idx_v)

        def scan_vreg(j, _):
            v = idx_v[pl.ds(j * L, L)]
            local = v - lo
            pos = (c * ICH + j * L) + lanes
            own = (local >= 0) & (local < R)
            _, lastocc = plsc.scan_count(v, mask=own)
            m = own & lastocc
            plsc.store_scatter(stamp, [local], pos, mask=m)
            return 0

        lax.fori_loop(0, ICH // L, scan_vreg, 0)
        return 0

    lax.fori_loop(0, N_ICH, scan_chunk, 0)

    def copy_group(g, _):
        for s in range(4):
            off = lo + (4 * g + s) * CB
            pltpu.async_copy(mem.at[pl.ds(off, CB)], cbs[s], isem)
        for s in range(4):
            off = lo + (4 * g + s) * CB
            pltpu.make_async_copy(mem.at[pl.ds(off, CB)], cbs[s], isem).wait()
            pltpu.async_copy(cbs[s], new_mem.at[pl.ds(off, CB)], osem)
        for s in range(4):
            off = lo + (4 * g + s) * CB
            pltpu.make_async_copy(cbs[s], new_mem.at[pl.ds(off, CB)], osem).wait()
        return 0

    lax.fori_loop(0, NB_FULL // 4, copy_group, 0)
    t_off = lo + NB_FULL * CB
    pltpu.sync_copy(mem.at[pl.ds(t_off, TAIL)], cb0.at[pl.ds(0, TAIL)])
    pltpu.sync_copy(cb0.at[pl.ds(0, TAIL)], new_mem.at[pl.ds(t_off, TAIL)])

    def extract(j, cursor):
        s = stamp[pl.ds(j * L, L)]
        m = s >= 0
        cnt = jnp.sum(m.astype(_i32))
        cur = jnp.minimum(cursor, CAP - L)
        rows = (lo + j * L) + lanes
        plsc.store_compressed(pos_list.at[pl.ds(cur, L)], s, mask=m)
        plsc.store_compressed(row_list.at[pl.ds(cur, L)], rows, mask=m)
        return cursor + cnt

    count = lax.fori_loop(0, SPAD // L, extract, jnp.asarray(0, _i32))
    count = jnp.minimum(count, CAP)

    w0 = pos_list[pl.ds(0, L)]
    r0 = row_list[pl.ds(0, L)]
    p0 = jnp.sum(jnp.where(lanes == 0, w0, 0))
    q0 = jnp.sum(jnp.where(lanes == 0, r0, 0))
    padp = jnp.full((L,), 1, _i32) * p0
    padq = jnp.full((L,), 1, _i32) * q0

    nblocks = (count + CB - 1) // CB
    cend = nblocks * CB
    j0 = (count // L) * L

    def pad_body(t, _):
        base = t * L
        keep = (base + lanes) < count
        wp = jnp.where(keep, pos_list[pl.ds(base, L)], padp)
        rp = jnp.where(keep, row_list[pl.ds(base, L)], padq)
        pos_list[pl.ds(base, L)] = wp
        row_list[pl.ds(base, L)] = rp
        return 0

    lax.fori_loop(j0 // L, cend // L, pad_body, 0)

    def move_group(g, _):
        for s in range(4):
            b = 4 * g + s

            @pl.when(b < nblocks)
            def _():
                pltpu.async_copy(
                    val.at[pos_list.at[pl.ds(b * CB, CB)]], cbs[s], isem
                )
        for s in range(4):
            b = 4 * g + s

            @pl.when(b < nblocks)
            def _():
                pltpu.make_async_copy(
                    val.at[pos_list.at[pl.ds(b * CB, CB)]], cbs[s], isem
                ).wait()
                pltpu.async_copy(
                    cbs[s], new_mem.at[row_list.at[pl.ds(b * CB, CB)]], osem
                )
        for s in range(4):
            b = 4 * g + s

            @pl.when(b < nblocks)
            def _():
                pltpu.make_async_copy(
                    cbs[s], new_mem.at[row_list.at[pl.ds(b * CB, CB)]], osem
                ).wait()
        return 0

    lax.fori_loop(0, (nblocks + 3) // 4, move_group, 0)

    base = wid * B_PER_W
    for g in range(B_PER_W // (4 * CB)):
        for s in range(4):
            off = base + (4 * g + s) * CB
            gslot = gidx.at[pl.ds(s * CB, CB)]
            pltpu.sync_copy(idx.at[pl.ds(off, CB)], gslot)
            pltpu.async_copy(mem.at[gslot], cbs[s], isem)
        for s in range(4):
            off = base + (4 * g + s) * CB
            gslot = gidx.at[pl.ds(s * CB, CB)]
            pltpu.make_async_copy(mem.at[gslot], cbs[s], isem).wait()
            pltpu.async_copy(cbs[s], pulled.at[pl.ds(off, CB)], osem)
        for s in range(4):
            off = base + (4 * g + s) * CB
            pltpu.make_async_copy(cbs[s], pulled.at[pl.ds(off, CB)], osem).wait()


def kernel(mem, idx, val):
    pulled, new_mem = _sc_pool(mem, idx, val)
    return pulled, new_mem


# speedup vs baseline: 2.1589x; 1.0699x over previous
import functools

import jax
import jax.numpy as jnp
from jax import lax
from jax.experimental import pallas as pl
from jax.experimental.pallas import tpu as pltpu
from jax.experimental.pallas import tpu_sc as plsc

NC = 2
NS = 16
NW = NC * NS
L = 16

B = 131072
D = 64
N = 1000000

B_PER_W = B // NW
R = N // NW
SPAD = 31264
ICH = 4096
N_ICH = B // ICH
CAP = 8192
CB = 128
NBUF = 8
NB_FULL = R // CB
TAIL = R - NB_FULL * CB

_mesh = plsc.VectorSubcoreMesh(
    core_axis_name="c", subcore_axis_name="s", num_cores=NC, num_subcores=NS
)

_i32 = jnp.int32
_f32 = jnp.float32


@functools.partial(
    pl.kernel,
    out_type=(
        jax.ShapeDtypeStruct((B, D), _f32),
        jax.ShapeDtypeStruct((N, D), _f32),
    ),
    mesh=_mesh,
    compiler_params=pltpu.CompilerParams(
        use_tc_tiling_on_sc=False, needs_layout_passes=False
    ),
    scratch_types=[
        pltpu.VMEM((SPAD,), _i32),
        pltpu.VMEM((ICH,), _i32),
        pltpu.VMEM((CAP,), _i32),
        pltpu.VMEM((CAP,), _i32),
    ]
    + [pltpu.VMEM((CB, D), _f32) for _ in range(NBUF)]
    + [
        pltpu.SemaphoreType.DMA,
        pltpu.SemaphoreType.DMA,
    ],
)
def _sc_pool(mem, idx, val, pulled, new_mem,
             stamp, idx_v, pos_list, row_list,
             b0, b1, b2, b3, b4, b5, b6, b7,
             isem, osem):
    cbs = (b0, b1, b2, b3, b4, b5, b6, b7)
    wid = lax.axis_index("s") * NC + lax.axis_index("c")
    lo = wid * R
    lanes = lax.iota(_i32, L)
    neg1 = jnp.full((L,), -1, _i32)

    def init_body(j, _):
        stamp[pl.ds(j * L, L)] = neg1
        return 0
    lax.fori_loop(0, SPAD // L, init_body, 0)

    def scan_chunk(c, _):
        @pl.when(c >= 1)
        def _():
            for s in range(NBUF):
                b = NBUF * (c - 1) + s

                @pl.when(b < NB_FULL)
                def _():
                    off = lo + b * CB
                    pltpu.make_async_copy(
                        cbs[s], new_mem.at[pl.ds(off, CB)], osem
                    ).wait()

        for s in range(NBUF):
            b = NBUF * c + s

            @pl.when(b < NB_FULL)
            def _():
                off = lo + b * CB
                pltpu.async_copy(mem.at[pl.ds(off, CB)], cbs[s], isem)

        pltpu.sync_copy(idx.at[pl.ds(c * ICH, ICH)], idx_v)

        def scan_vreg(j, _):
            v = idx_v[pl.ds(j * L, L)]
            local = v - lo
            pos = (c * ICH + j * L) + lanes
            own = (local >= 0) & (local < R)
            _, lastocc = plsc.scan_count(v, mask=own)
            m = own & lastocc
            plsc.store_scatter(stamp, [local], pos, mask=m)
            return 0

        lax.fori_loop(0, ICH // L, scan_vreg, 0)

        for s in range(NBUF):
            b = NBUF * c + s

            @pl.when(b < NB_FULL)
            def _():
                off = lo + b * CB
                pltpu.make_async_copy(
                    mem.at[pl.ds(off, CB)], cbs[s], isem
                ).wait()
                pltpu.async_copy(cbs[s], new_mem.at[pl.ds(off, CB)], osem)
        return 0

    lax.fori_loop(0, N_ICH, scan_chunk, 0)
    for s in range(NBUF):
        b = NBUF * (N_ICH - 1) + s

        @pl.when(b < NB_FULL)
        def _():
            off = lo + b * CB
            pltpu.make_async_copy(
                cbs[s], new_mem.at[pl.ds(off, CB)], osem
            ).wait()
    t_off = lo + NB_FULL * CB
    pltpu.sync_copy(mem.at[pl.ds(t_off, TAIL)], b0.at[pl.ds(0, TAIL)])
    pltpu.sync_copy(b0.at[pl.ds(0, TAIL)], new_mem.at[pl.ds(t_off, TAIL)])

    def extract(j, cursor):
        s = stamp[pl.ds(j * L, L)]
        m = s >= 0
        cnt = jnp.sum(m.astype(_i32))
        cur = jnp.minimum(cursor, CAP - L)
        rows = (lo + j * L) + lanes
        plsc.store_compressed(pos_list.at[pl.ds(cur, L)], s, mask=m)
        plsc.store_compressed(row_list.at[pl.ds(cur, L)], rows, mask=m)
        return cursor + cnt

    count = lax.fori_loop(0, SPAD // L, extract, jnp.asarray(0, _i32))
    count = jnp.minimum(count, CAP)

    w0 = pos_list[pl.ds(0, L)]
    r0 = row_list[pl.ds(0, L)]
    p0 = jnp.sum(jnp.where(lanes == 0, w0, 0))
    q0 = jnp.sum(jnp.where(lanes == 0, r0, 0))
    padp = jnp.full((L,), 1, _i32) * p0
    padq = jnp.full((L,), 1, _i32) * q0

    nblocks = (count + CB - 1) // CB
    cend = nblocks * CB
    j0 = (count // L) * L

    def pad_body(t, _):
        base = t * L
        keep = (base + lanes) < count
        wp = jnp.where(keep, pos_list[pl.ds(base, L)], padp)
        rp = jnp.where(keep, row_list[pl.ds(base, L)], padq)
        pos_list[pl.ds(base, L)] = wp
        row_list[pl.ds(base, L)] = rp
        return 0

    lax.fori_loop(j0 // L, cend // L, pad_body, 0)

    def move_group(g, _):
        for s in range(NBUF):
            b = NBUF * g + s

            @pl.when(b < nblocks)
            def _():
                pltpu.async_copy(
                    val.at[pos_list.at[pl.ds(b * CB, CB)]], cbs[s], isem
                )
        for s in range(NBUF):
            b = NBUF * g + s

            @pl.when(b < nblocks)
            def _():
                pltpu.make_async_copy(
                    val.at[pos_list.at[pl.ds(b * CB, CB)]], cbs[s], isem
                ).wait()
                pltpu.async_copy(
                    cbs[s], new_mem.at[row_list.at[pl.ds(b * CB, CB)]], osem
                )
        for s in range(NBUF):
            b = NBUF * g + s

            @pl.when(b < nblocks)
            def _():
                pltpu.make_async_copy(
                    cbs[s], new_mem.at[row_list.at[pl.ds(b * CB, CB)]], osem
                ).wait()
        return 0

    lax.fori_loop(0, (nblocks + NBUF - 1) // NBUF, move_group, 0)

    base = wid * B_PER_W
    pltpu.sync_copy(idx.at[pl.ds(base, B_PER_W)], idx_v.at[pl.ds(0, B_PER_W)])
    for g in range(B_PER_W // (NBUF * CB)):
        for s in range(NBUF):
            k = (NBUF * g + s) * CB
            pltpu.async_copy(
                mem.at[idx_v.at[pl.ds(k, CB)]], cbs[s], isem
            )
        for s in range(NBUF):
            k = (NBUF * g + s) * CB
            pltpu.make_async_copy(
                mem.at[idx_v.at[pl.ds(k, CB)]], cbs[s], isem
            ).wait()
            pltpu.async_copy(cbs[s], pulled.at[pl.ds(base + k, CB)], osem)
        for s in range(NBUF):
            k = (NBUF * g + s) * CB
            pltpu.make_async_copy(
                cbs[s], pulled.at[pl.ds(base + k, CB)], osem
            ).wait()


def kernel(mem, idx, val):
    pulled, new_mem = _sc_pool(mem, idx, val)
    return pulled, new_mem


# speedup vs baseline: 2.1620x; 1.0014x over previous
import functools

import jax
import jax.numpy as jnp
from jax import lax
from jax.experimental import pallas as pl
from jax.experimental.pallas import tpu as pltpu
from jax.experimental.pallas import tpu_sc as plsc

NC = 2
NS = 16
NW = NC * NS
L = 16

B = 131072
D = 64
N = 1000000

B_PER_W = B // NW
R = N // NW
SPAD = 31264
ICH = 4096
N_ICH = B // ICH
CAP = 8192
CB = 128
NBUF = 8
NB_FULL = R // CB
TAIL = R - NB_FULL * CB

_mesh = plsc.VectorSubcoreMesh(
    core_axis_name="c", subcore_axis_name="s", num_cores=NC, num_subcores=NS
)

_i32 = jnp.int32
_f32 = jnp.float32


@functools.partial(
    pl.kernel,
    out_type=(
        jax.ShapeDtypeStruct((B, D), _f32),
        jax.ShapeDtypeStruct((N, D), _f32),
    ),
    mesh=_mesh,
    compiler_params=pltpu.CompilerParams(
        use_tc_tiling_on_sc=False, needs_layout_passes=False
    ),
    scratch_types=[
        pltpu.VMEM((SPAD,), _i32),
        pltpu.VMEM((ICH,), _i32),
        pltpu.VMEM((CAP,), _i32),
        pltpu.VMEM((CAP,), _i32),
    ]
    + [pltpu.VMEM((CB, D), _f32) for _ in range(NBUF)]
    + [
        pltpu.SemaphoreType.DMA,
        pltpu.SemaphoreType.DMA,
    ],
)
def _sc_pool(mem, idx, val, pulled, new_mem,
             stamp, idx_v, pos_list, row_list,
             b0, b1, b2, b3, b4, b5, b6, b7,
             isem, osem):
    cbs = (b0, b1, b2, b3, b4, b5, b6, b7)
    wid = lax.axis_index("s") * NC + lax.axis_index("c")
    lo = wid * R
    lanes = lax.iota(_i32, L)
    neg1 = jnp.full((L,), -1, _i32)

    def init_body(j, _):
        stamp[pl.ds(j * L, L)] = neg1
        return 0
    lax.fori_loop(0, SPAD // L, init_body, 0)

    def scan_chunk(c, _):
        @pl.when(c >= 1)
        def _():
            for s in range(NBUF):
                b = NBUF * (c - 1) + s

                @pl.when(b < NB_FULL)
                def _():
                    off = lo + b * CB
                    pltpu.make_async_copy(
                        cbs[s], new_mem.at[pl.ds(off, CB)], osem
                    ).wait()

        for s in range(NBUF):
            b = NBUF * c + s

            @pl.when(b < NB_FULL)
            def _():
                off = lo + b * CB
                pltpu.async_copy(mem.at[pl.ds(off, CB)], cbs[s], isem)

        pltpu.sync_copy(idx.at[pl.ds(c * ICH, ICH)], idx_v)

        def scan_vreg(j, _):
            for u in range(4):
                v = idx_v[pl.ds((j * 4 + u) * L, L)]
                local = v - lo
                pos = (c * ICH + (j * 4 + u) * L) + lanes
                own = (local >= 0) & (local < R)
                _, lastocc = plsc.scan_count(v, mask=own)
                m = own & lastocc
                plsc.store_scatter(stamp, [local], pos, mask=m)
            return 0

        lax.fori_loop(0, ICH // L // 4, scan_vreg, 0)

        for s in range(NBUF):
            b = NBUF * c + s

            @pl.when(b < NB_FULL)
            def _():
                off = lo + b * CB
                pltpu.make_async_copy(
                    mem.at[pl.ds(off, CB)], cbs[s], isem
                ).wait()
                pltpu.async_copy(cbs[s], new_mem.at[pl.ds(off, CB)], osem)
        return 0

    lax.fori_loop(0, N_ICH, scan_chunk, 0)
    for s in range(NBUF):
        b = NBUF * (N_ICH - 1) + s

        @pl.when(b < NB_FULL)
        def _():
            off = lo + b * CB
            pltpu.make_async_copy(
                cbs[s], new_mem.at[pl.ds(off, CB)], osem
            ).wait()
    t_off = lo + NB_FULL * CB
    pltpu.sync_copy(mem.at[pl.ds(t_off, TAIL)], b0.at[pl.ds(0, TAIL)])
    pltpu.sync_copy(b0.at[pl.ds(0, TAIL)], new_mem.at[pl.ds(t_off, TAIL)])

    def extract(j, cursor):
        s = stamp[pl.ds(j * L, L)]
        m = s >= 0
        cnt = jnp.sum(m.astype(_i32))
        cur = jnp.minimum(cursor, CAP - L)
        rows = (lo + j * L) + lanes
        plsc.store_compressed(pos_list.at[pl.ds(cur, L)], s, mask=m)
        plsc.store_compressed(row_list.at[pl.ds(cur, L)], rows, mask=m)
        return cursor + cnt

    count = lax.fori_loop(0, SPAD // L, extract, jnp.asarray(0, _i32))
    count = jnp.minimum(count, CAP)

    w0 = pos_list[pl.ds(0, L)]
    r0 = row_list[pl.ds(0, L)]
    p0 = jnp.sum(jnp.where(lanes == 0, w0, 0))
    q0 = jnp.sum(jnp.where(lanes == 0, r0, 0))
    padp = jnp.full((L,), 1, _i32) * p0
    padq = jnp.full((L,), 1, _i32) * q0

    nblocks = (count + CB - 1) // CB
    cend = nblocks * CB
    j0 = (count // L) * L

    def pad_body(t, _):
        base = t * L
        keep = (base + lanes) < count
        wp = jnp.where(keep, pos_list[pl.ds(base, L)], padp)
        rp = jnp.where(keep, row_list[pl.ds(base, L)], padq)
        pos_list[pl.ds(base, L)] = wp
        row_list[pl.ds(base, L)] = rp
        return 0

    lax.fori_loop(j0 // L, cend // L, pad_body, 0)

    base = wid * B_PER_W
    pltpu.sync_copy(idx.at[pl.ds(base, B_PER_W)], idx_v.at[pl.ds(0, B_PER_W)])
    NGG = B_PER_W // (4 * CB)
    ncg = (nblocks + 3) // 4

    def cg_group(g, _):
        for s in range(4):
            b = 4 * g + s

            @pl.when(b < nblocks)
            def _():
                pltpu.async_copy(
                    val.at[pos_list.at[pl.ds(b * CB, CB)]], cbs[s], isem
                )

        @pl.when(g < NGG)
        def _():
            for s in range(4):
                k = (4 * g + s) * CB
                pltpu.async_copy(
                    mem.at[idx_v.at[pl.ds(k, CB)]], cbs[4 + s], isem
                )

        for s in range(4):
            b = 4 * g + s

            @pl.when(b < nblocks)
            def _():
                pltpu.make_async_copy(
                    val.at[pos_list.at[pl.ds(b * CB, CB)]], cbs[s], isem
                ).wait()
                pltpu.async_copy(
                    cbs[s], new_mem.at[row_list.at[pl.ds(b * CB, CB)]], osem
                )

        @pl.when(g < NGG)
        def _():
            for s in range(4):
                k = (4 * g + s) * CB
                pltpu.make_async_copy(
                    mem.at[idx_v.at[pl.ds(k, CB)]], cbs[4 + s], isem
                ).wait()
                pltpu.async_copy(
                    cbs[4 + s], pulled.at[pl.ds(base + k, CB)], osem
                )

        for s in range(4):
            b = 4 * g + s

            @pl.when(b < nblocks)
            def _():
                pltpu.make_async_copy(
                    cbs[s], new_mem.at[row_list.at[pl.ds(b * CB, CB)]], osem
                ).wait()

        @pl.when(g < NGG)
        def _():
            for s in range(4):
                k = (4 * g + s) * CB
                pltpu.make_async_copy(
                    cbs[4 + s], pulled.at[pl.ds(base + k, CB)], osem
                ).wait()
        return 0

    lax.fori_loop(0, jnp.maximum(ncg, NGG), cg_group, 0)


def kernel(mem, idx, val):
    pulled, new_mem = _sc_pool(mem, idx, val)
    return pulled, new_mem


# speedup vs baseline: 2.2187x; 1.0262x over previous
import functools

import jax
import jax.numpy as jnp
from jax import lax
from jax.experimental import pallas as pl
from jax.experimental.pallas import tpu as pltpu
from jax.experimental.pallas import tpu_sc as plsc

NC = 2
NS = 16
NW = NC * NS
L = 16

B = 131072
D = 64
N = 1000000

B_PER_W = B // NW
R = N // NW
SPAD = 31264
ICH = 4096
N_ICH = B // ICH
CAP = 8192
CB = 128
NBUF = 8
NB_FULL = R // CB
TAIL = R - NB_FULL * CB

_mesh = plsc.VectorSubcoreMesh(
    core_axis_name="c", subcore_axis_name="s", num_cores=NC, num_subcores=NS
)

_params = pltpu.CompilerParams(
    use_tc_tiling_on_sc=False, needs_layout_passes=False
)

_i32 = jnp.int32
_f32 = jnp.float32


@functools.partial(
    pl.kernel,
    out_type=jax.ShapeDtypeStruct((N, D), _f32),
    mesh=_mesh,
    compiler_params=_params,
    scratch_types=[
        pltpu.VMEM((SPAD,), _i32),
        pltpu.VMEM((ICH,), _i32),
        pltpu.VMEM((CAP,), _i32),
        pltpu.VMEM((CAP,), _i32),
    ]
    + [pltpu.VMEM((CB, D), _f32) for _ in range(NBUF)]
    + [
        pltpu.SemaphoreType.DMA,
        pltpu.SemaphoreType.DMA,
    ],
)
def _sc_scatter(mem, idx, val, new_mem,
                stamp, idx_v, pos_list, row_list,
                b0, b1, b2, b3, b4, b5, b6, b7,
                isem, osem):
    cbs = (b0, b1, b2, b3, b4, b5, b6, b7)
    wid = lax.axis_index("s") * NC + lax.axis_index("c")
    lo = wid * R
    lanes = lax.iota(_i32, L)
    neg1 = jnp.full((L,), -1, _i32)

    def init_body(j, _):
        stamp[pl.ds(j * L, L)] = neg1
        return 0
    lax.fori_loop(0, SPAD // L, init_body, 0)

    def scan_chunk(c, _):
        @pl.when(c >= 1)
        def _():
            for s in range(NBUF):
                b = NBUF * (c - 1) + s

                @pl.when(b < NB_FULL)
                def _():
                    off = lo + b * CB
                    pltpu.make_async_copy(
                        cbs[s], new_mem.at[pl.ds(off, CB)], osem
                    ).wait()

        for s in range(NBUF):
            b = NBUF * c + s

            @pl.when(b < NB_FULL)
            def _():
                off = lo + b * CB
                pltpu.async_copy(mem.at[pl.ds(off, CB)], cbs[s], isem)

        pltpu.sync_copy(idx.at[pl.ds(c * ICH, ICH)], idx_v)

        def scan_vreg(j, _):
            for u in range(4):
                v = idx_v[pl.ds((j * 4 + u) * L, L)]
                local = v - lo
                pos = (c * ICH + (j * 4 + u) * L) + lanes
                own = (local >= 0) & (local < R)
                _, lastocc = plsc.scan_count(v, mask=own)
                m = own & lastocc
                plsc.store_scatter(stamp, [local], pos, mask=m)
            return 0

        lax.fori_loop(0, ICH // L // 4, scan_vreg, 0)

        for s in range(NBUF):
            b = NBUF * c + s

            @pl.when(b < NB_FULL)
            def _():
                off = lo + b * CB
                pltpu.make_async_copy(
                    mem.at[pl.ds(off, CB)], cbs[s], isem
                ).wait()
                pltpu.async_copy(cbs[s], new_mem.at[pl.ds(off, CB)], osem)
        return 0

    lax.fori_loop(0, N_ICH, scan_chunk, 0)
    for s in range(NBUF):
        b = NBUF * (N_ICH - 1) + s

        @pl.when(b < NB_FULL)
        def _():
            off = lo + b * CB
            pltpu.make_async_copy(
                cbs[s], new_mem.at[pl.ds(off, CB)], osem
            ).wait()
    t_off = lo + NB_FULL * CB
    pltpu.sync_copy(mem.at[pl.ds(t_off, TAIL)], b0.at[pl.ds(0, TAIL)])
    pltpu.sync_copy(b0.at[pl.ds(0, TAIL)], new_mem.at[pl.ds(t_off, TAIL)])

    def extract(j, cursor):
        s = stamp[pl.ds(j * L, L)]
        m = s >= 0
        cnt = jnp.sum(m.astype(_i32))
        cur = jnp.minimum(cursor, CAP - L)
        rows = (lo + j * L) + lanes
        plsc.store_compressed(pos_list.at[pl.ds(cur, L)], s, mask=m)
        plsc.store_compressed(row_list.at[pl.ds(cur, L)], rows, mask=m)
        return cursor + cnt

    count = lax.fori_loop(0, SPAD // L, extract, jnp.asarray(0, _i32))
    count = jnp.minimum(count, CAP)

    w0 = pos_list[pl.ds(0, L)]
    r0 = row_list[pl.ds(0, L)]
    p0 = jnp.sum(jnp.where(lanes == 0, w0, 0))
    q0 = jnp.sum(jnp.where(lanes == 0, r0, 0))
    padp = jnp.full((L,), 1, _i32) * p0
    padq = jnp.full((L,), 1, _i32) * q0

    nblocks = (count + CB - 1) // CB
    cend = nblocks * CB
    j0 = (count // L) * L

    def pad_body(t, _):
        base = t * L
        keep = (base + lanes) < count
        wp = jnp.where(keep, pos_list[pl.ds(base, L)], padp)
        rp = jnp.where(keep, row_list[pl.ds(base, L)], padq)
        pos_list[pl.ds(base, L)] = wp
        row_list[pl.ds(base, L)] = rp
        return 0

    lax.fori_loop(j0 // L, cend // L, pad_body, 0)

    def move_group(g, _):
        for s in range(NBUF):
            b = NBUF * g + s

            @pl.when(b < nblocks)
            def _():
                pltpu.async_copy(
                    val.at[pos_list.at[pl.ds(b * CB, CB)]], cbs[s], isem
                )
        for s in range(NBUF):
            b = NBUF * g + s

            @pl.when(b < nblocks)
            def _():
                pltpu.make_async_copy(
                    val.at[pos_list.at[pl.ds(b * CB, CB)]], cbs[s], isem
                ).wait()
                pltpu.async_copy(
                    cbs[s], new_mem.at[row_list.at[pl.ds(b * CB, CB)]], osem
                )
        for s in range(NBUF):
            b = NBUF * g + s

            @pl.when(b < nblocks)
            def _():
                pltpu.make_async_copy(
                    cbs[s], new_mem.at[row_list.at[pl.ds(b * CB, CB)]], osem
                ).wait()
        return 0

    lax.fori_loop(0, (nblocks + NBUF - 1) // NBUF, move_group, 0)


@functools.partial(
    pl.kernel,
    out_type=jax.ShapeDtypeStruct((B, D), _f32),
    mesh=_mesh,
    compiler_params=_params,
    scratch_types=[
        pltpu.VMEM((B_PER_W,), _i32),
    ]
    + [pltpu.VMEM((CB, D), _f32) for _ in range(NBUF)]
    + [
        pltpu.SemaphoreType.DMA,
        pltpu.SemaphoreType.DMA,
    ],
)
def _sc_gather(mem, idx, pulled,
               idx_v, b0, b1, b2, b3, b4, b5, b6, b7, isem, osem):
    cbs = (b0, b1, b2, b3, b4, b5, b6, b7)
    wid = lax.axis_index("s") * NC + lax.axis_index("c")
    base = wid * B_PER_W
    pltpu.sync_copy(idx.at[pl.ds(base, B_PER_W)], idx_v)
    for g in range(B_PER_W // (NBUF * CB)):
        for s in range(NBUF):
            k = (NBUF * g + s) * CB
            pltpu.async_copy(mem.at[idx_v.at[pl.ds(k, CB)]], cbs[s], isem)
        for s in range(NBUF):
            k = (NBUF * g + s) * CB
            pltpu.make_async_copy(
                mem.at[idx_v.at[pl.ds(k, CB)]], cbs[s], isem
            ).wait()
            pltpu.async_copy(cbs[s], pulled.at[pl.ds(base + k, CB)], osem)
        for s in range(NBUF):
            k = (NBUF * g + s) * CB
            pltpu.make_async_copy(
                cbs[s], pulled.at[pl.ds(base + k, CB)], osem
            ).wait()


def kernel(mem, idx, val):
    new_mem = _sc_scatter(mem, idx, val)
    pulled = _sc_gather(mem, idx)
    return pulled, new_mem
